# Initial kernel scaffold; baseline (speedup 1.0000x reference)
import jax
import jax.numpy as jnp
from jax.experimental import pallas as pl


def kernel(input, s_1, weights, bias, alpha_param):
    raise NotImplementedError("write your pallas kernel here")


# speedup vs baseline: 4.2002x; 4.2002x over previous
import jax
import jax.numpy as jnp
from jax.experimental import pallas as pl
from jax.experimental.pallas import tpu as pltpu

_B, _C, _T, _F = 16, 2, 2000, 257
_L = 2 * _F
_TC = 400


def _ema_norm_kernel(x_ref, s1_ref, ap_ref, w_ref, b_ref,
                     res_ref, sfin_ref, smooth_ref, carry_ref):
    t = pl.program_id(1)

    @pl.when(t == 0)
    def _():
        carry_ref[0, :] = s1_ref[0, 0, :]

    x0 = x_ref[0, 0]
    x1 = x_ref[0, 1]

    lane = jax.lax.broadcasted_iota(jnp.int32, (_TC, _L), 1)
    even = (lane % 2) == 0

    def pair_sum(x):
        p = x * x
        return jnp.where(even,
                         p + jnp.roll(p, -1, axis=1),
                         p + jnp.roll(p, 1, axis=1))

    d2_0 = pair_sum(x0)
    d2_1 = pair_sum(x1)

    a = jax.nn.sigmoid(ap_ref[0, :])
    la = jnp.log1p(-a)

    y = d2_0 * a
    riota_i = jax.lax.broadcasted_iota(jnp.int32, (_TC, _L), 0)
    riota = riota_i.astype(jnp.float32)
    d = 1
    while d < _TC:
        dec = jnp.exp(la * float(d))
        y = y + jnp.where(riota_i >= d,
                          dec * jnp.roll(y, d, axis=0), 0.0)
        d *= 2

    s = y + jnp.exp(la * (riota + 1.0)) * carry_ref[0, :]
    carry_ref[0, :] = s[_TC - 1, :]
    sfin_ref[0, 0, :] = s[_TC - 1, :]

    smooth = jnp.sqrt(s)
    smooth_ref[0] = smooth

    res_ref[0, 0] = x0 / (smooth + 1e-8) * w_ref[0, :] + b_ref[0, :]
    res_ref[0, 1] = x1 / (jnp.sqrt(d2_1) + 1e-8) * w_ref[1, :] + b_ref[1, :]


def kernel(input, s_1, weights, bias, alpha_param):
    B, C, T, F = _B, _C, _T, _F
    L = _L

    x = input.reshape(B, C, T, L)
    s1_i = jnp.repeat(s_1.reshape(B, 1, F), 2, axis=-1)
    ap_i = jnp.repeat(alpha_param.reshape(1, F), 2, axis=-1)
    w_i = jnp.repeat(weights.reshape(C, F), 2, axis=-1)
    b_i = jnp.repeat(bias.reshape(C, F), 2, axis=-1)

    nt = T // _TC
    res_i, sfin_i, smooth_i = pl.pallas_call(
        _ema_norm_kernel,
        grid=(B, nt),
        in_specs=[
            pl.BlockSpec((1, C, _TC, L), lambda b, t: (b, 0, t, 0)),
            pl.BlockSpec((1, 1, L), lambda b, t: (b, 0, 0)),
            pl.BlockSpec((1, L), lambda b, t: (0, 0)),
            pl.BlockSpec((C, L), lambda b, t: (0, 0)),
            pl.BlockSpec((C, L), lambda b, t: (0, 0)),
        ],
        out_specs=[
            pl.BlockSpec((1, C, _TC, L), lambda b, t: (b, 0, t, 0)),
            pl.BlockSpec((1, 1, L), lambda b, t: (b, 0, 0)),
            pl.BlockSpec((1, _TC, L), lambda b, t: (b, t, 0)),
        ],
        out_shape=[
            jax.ShapeDtypeStruct((B, C, T, L), jnp.float32),
            jax.ShapeDtypeStruct((B, 1, L), jnp.float32),
            jax.ShapeDtypeStruct((B, T, L), jnp.float32),
        ],
        scratch_shapes=[pltpu.VMEM((1, L), jnp.float32)],
        compiler_params=pltpu.CompilerParams(
            dimension_semantics=("parallel", "arbitrary"),
        ),
        name="ema_norm",
    )(x, s1_i, ap_i, w_i, b_i)

    res = res_i.reshape(B, C, T, F, 2)
    s_final = sfin_i.reshape(B, 1, F, 2)[..., 0:1]
    smooth_data = smooth_i.reshape(B, T, F, 2)[..., 0]
    smooth_data = smooth_data.reshape(B, 1, T, F, 1)
    return res, s_final, smooth_data
